# Initial kernel scaffold; baseline (speedup 1.0000x reference)
import jax
import jax.numpy as jnp
from jax.experimental import pallas as pl


def kernel(x, edge_index, edge_attr, W1, b1, W2, b2):
    raise NotImplementedError("write your pallas kernel here")


# speedup vs baseline: 2.2287x; 2.2287x over previous
import functools

import jax
import jax.numpy as jnp
from jax import lax
from jax.experimental import pallas as pl
from jax.experimental.pallas import tpu as pltpu
from jax.experimental.pallas import tpu_sc as plsc

_CHUNK = 128


def _sc_gather(src, dst, x):
    n_edges = src.shape[0]
    n_chunks = n_edges // _CHUNK
    info = plsc.get_sparse_core_info()
    nc, ns = info.num_cores, info.num_subcores
    nw = nc * ns
    d_feat = x.shape[1]
    mesh = plsc.VectorSubcoreMesh(core_axis_name="c", subcore_axis_name="s")

    @functools.partial(
        pl.kernel,
        mesh=mesh,
        out_type=(
            jax.ShapeDtypeStruct((n_edges, d_feat), jnp.float32),
            jax.ShapeDtypeStruct((n_edges, d_feat), jnp.float32),
        ),
        scratch_types=[
            pltpu.VMEM((_CHUNK,), jnp.int32),
            pltpu.VMEM((_CHUNK,), jnp.int32),
            pltpu.VMEM((_CHUNK, d_feat), jnp.float32),
            pltpu.VMEM((_CHUNK, d_feat), jnp.float32),
            pltpu.SemaphoreType.DMA,
            pltpu.SemaphoreType.DMA,
        ],
    )
    def gather_kernel(src_hbm, dst_hbm, x_hbm, gs_hbm, gd_hbm,
                      idxs, idxd, bufs, bufd, sems, semd):
        wid = lax.axis_index("s") * nc + lax.axis_index("c")
        n_local = (n_chunks - wid + nw - 1) // nw

        def body(i, carry):
            base = (wid + i * nw) * _CHUNK
            pltpu.sync_copy(src_hbm.at[pl.ds(base, _CHUNK)], idxs)
            pltpu.sync_copy(dst_hbm.at[pl.ds(base, _CHUNK)], idxd)
            cs = pltpu.async_copy(x_hbm.at[idxs], bufs, sems)
            cd = pltpu.async_copy(x_hbm.at[idxd], bufd, semd)
            cs.wait()
            cd.wait()
            pltpu.sync_copy(bufs, gs_hbm.at[pl.ds(base, _CHUNK)])
            pltpu.sync_copy(bufd, gd_hbm.at[pl.ds(base, _CHUNK)])
            return carry

        lax.fori_loop(0, n_local, body, 0)

    return gather_kernel(src, dst, x)


def _mlp(gs, gd, ea, w1s, w1d, w1e, b1, w2, b2):
    n_edges = gs.shape[0]
    rows = 1280
    d_feat = gs.shape[1]
    d_edge = ea.shape[1]
    d_hid = w2.shape[0]
    d_out = w2.shape[1]

    def body(gs_ref, gd_ref, ea_ref, w1s_ref, w1d_ref, w1e_ref,
             b1_ref, w2_ref, b2_ref, o_ref):
        z = (jnp.dot(gs_ref[...], w1s_ref[...], preferred_element_type=jnp.float32)
             + jnp.dot(gd_ref[...], w1d_ref[...], preferred_element_type=jnp.float32)
             + jnp.dot(ea_ref[...], w1e_ref[...], preferred_element_type=jnp.float32)
             + b1_ref[...])
        h = jnp.maximum(z, 0.0)
        o_ref[...] = (jnp.dot(h, w2_ref[...], preferred_element_type=jnp.float32)
                      + b2_ref[...])

    return pl.pallas_call(
        body,
        grid=(n_edges // rows,),
        in_specs=[
            pl.BlockSpec((rows, d_feat), lambda i: (i, 0)),
            pl.BlockSpec((rows, d_feat), lambda i: (i, 0)),
            pl.BlockSpec((rows, d_edge), lambda i: (i, 0)),
            pl.BlockSpec((d_feat, d_hid), lambda i: (0, 0)),
            pl.BlockSpec((d_feat, d_hid), lambda i: (0, 0)),
            pl.BlockSpec((d_edge, d_hid), lambda i: (0, 0)),
            pl.BlockSpec((1, d_hid), lambda i: (0, 0)),
            pl.BlockSpec((d_hid, d_out), lambda i: (0, 0)),
            pl.BlockSpec((1, d_out), lambda i: (0, 0)),
        ],
        out_specs=pl.BlockSpec((rows, d_out), lambda i: (i, 0)),
        out_shape=jax.ShapeDtypeStruct((n_edges, d_out), jnp.float32),
    )(gs, gd, ea, w1s, w1d, w1e, b1.reshape(1, -1), w2, b2.reshape(1, -1))


def kernel(x, edge_index, edge_attr, W1, b1, W2, b2):
    ei = edge_index.astype(jnp.int32)
    src = ei[:, 0]
    dst = ei[:, 1]
    gs, gd = _sc_gather(src, dst, x)
    d_feat = x.shape[1]
    return _mlp(gs, gd, edge_attr,
                W1[:d_feat], W1[d_feat:2 * d_feat], W1[2 * d_feat:],
                b1, W2, b2)
